# baseline (speedup 1.0000x reference)
import functools

import jax
import jax.numpy as jnp
from jax import lax
from jax.experimental import pallas as pl
from jax.experimental.pallas import tpu as pltpu
from jax.experimental.pallas import tpu_sc as plsc

NUM_EMBEDDINGS = 1000000
D = 64
B = 4096
H = 200
EPS = 1e-5

N = B * H
NC, NS = 2, 16
NW = NC * NS
PER_W = N // NW
CHUNK = 512
IDX_ROWS = CHUNK // 128
N_CHUNKS = PER_W // CHUNK
GROUPS = CHUNK // 16


def _rsqrt(x):
    i = lax.bitcast_convert_type(x, jnp.int32)
    i = jnp.int32(0x5F3759DF) - lax.shift_right_logical(i, 1)
    y = lax.bitcast_convert_type(i, jnp.float32)
    for _ in range(3):
        y = y * (1.5 - 0.5 * x * y * y)
    return y


def _layernorm_chunk(rows_v, gam, bet):
    lanes = lax.iota(jnp.int32, 16)

    def group_body(grp, _):
        row_ids = grp * 16 + lanes
        s = jnp.zeros((16,), jnp.float32)
        q = jnp.zeros((16,), jnp.float32)
        for j in range(D):
            col = jnp.full((16,), j, jnp.int32)
            g = plsc.load_gather(rows_v, [row_ids, col])
            s = s + g
            q = q + g * g
        mean = s * (1.0 / D)
        var = q * (1.0 / D) - mean * mean
        inv = _rsqrt(var + EPS)
        for j in range(D):
            col = jnp.full((16,), j, jnp.int32)
            idxj = jnp.full((16,), j % 16, jnp.int32)
            g = plsc.load_gather(rows_v, [row_ids, col])
            gj = jnp.take_along_axis(gam[j // 16], idxj, axis=0)
            bj = jnp.take_along_axis(bet[j // 16], idxj, axis=0)
            o = (g - mean) * inv * gj + bj
            plsc.store_scatter(rows_v, [row_ids, col], o)
        return None

    lax.fori_loop(0, GROUPS, group_body, None)


def _body(x_hbm, table_hbm, gamma_hbm, beta_hbm, out_hbm,
          idx_v, rows_v, gb_v, gsem):
    wid = lax.axis_index("s") * NC + lax.axis_index("c")
    base = wid * PER_W
    xrow0 = wid * (PER_W // 128)

    pltpu.sync_copy(gamma_hbm, gb_v.at[0])
    pltpu.sync_copy(beta_hbm, gb_v.at[1])
    gam = [gb_v[0, pl.ds(16 * k, 16)] for k in range(4)]
    bet = [gb_v[1, pl.ds(16 * k, 16)] for k in range(4)]

    def chunk_body(c, _):
        pltpu.sync_copy(x_hbm.at[pl.ds(xrow0 + c * IDX_ROWS, IDX_ROWS)], idx_v)
        copies = [
            pltpu.async_copy(
                table_hbm.at[idx_v.at[j]],
                rows_v.at[pl.ds(j * 128, 128)],
                gsem,
            )
            for j in range(IDX_ROWS)
        ]
        for cp in copies:
            cp.wait()
        _layernorm_chunk(rows_v, gam, bet)
        pltpu.sync_copy(rows_v, out_hbm.at[pl.ds(base + c * CHUNK, CHUNK)])
        return None

    lax.fori_loop(0, N_CHUNKS, chunk_body, None)


@jax.jit
def _run(x2d, table, gamma, beta):
    mesh = plsc.VectorSubcoreMesh(core_axis_name="c", subcore_axis_name="s")
    kern = pl.kernel(
        _body,
        out_type=jax.ShapeDtypeStruct((N, D), jnp.float32),
        mesh=mesh,
        scratch_types=[
            pltpu.VMEM((IDX_ROWS, 128), jnp.int32),
            pltpu.VMEM((CHUNK, D), jnp.float32),
            pltpu.VMEM((2, D), jnp.float32),
            pltpu.SemaphoreType.DMA,
        ],
        compiler_params=pltpu.CompilerParams(
            use_tc_tiling_on_sc=False, needs_layout_passes=False
        ),
    )
    return kern(x2d, table, gamma, beta)


def kernel(x, table, gamma, beta):
    x2d = x.reshape(N // 128, 128).astype(jnp.int32)
    out = _run(x2d, table, gamma, beta)
    return out.reshape(B, H, D)


# speedup vs baseline: 3.4017x; 3.4017x over previous
import functools

import jax
import jax.numpy as jnp
from jax import lax
from jax.experimental import pallas as pl
from jax.experimental.pallas import tpu as pltpu
from jax.experimental.pallas import tpu_sc as plsc

NUM_EMBEDDINGS = 1000000
D = 64
B = 4096
H = 200
EPS = 1e-5

N = B * H
NC, NS = 2, 16
NW = NC * NS
PER_W = N // NW
IDXR_W = PER_W // 128
CHUNK = 256
IDX_ROWS = CHUNK // 128
N_CHUNKS = PER_W // CHUNK
NH = N_CHUNKS // 2
GROUPS = CHUNK // 16


def _rsqrt(x):
    i = lax.bitcast_convert_type(x, jnp.int32)
    i = jnp.int32(0x5F3759DF) - lax.shift_right_logical(i, 1)
    y = lax.bitcast_convert_type(i, jnp.float32)
    for _ in range(3):
        y = y * (1.5 - 0.5 * x * y * y)
    return y


def _body(x_hbm, table_hbm, gamma_hbm, beta_hbm, out_hbm,
          idx_all, rows0, rows1, outb0, outb1, gb_v,
          gsem0, gsem1, osem0, osem1):
    wid = lax.axis_index("s") * NC + lax.axis_index("c")
    base = wid * PER_W
    xrow0 = wid * IDXR_W

    pltpu.sync_copy(x_hbm.at[pl.ds(xrow0, IDXR_W)], idx_all)
    pltpu.sync_copy(gamma_hbm, gb_v.at[0])
    pltpu.sync_copy(beta_hbm, gb_v.at[1])
    gam = [gb_v[0, pl.ds(16 * k, 16)] for k in range(4)]
    bet = [gb_v[1, pl.ds(16 * k, 16)] for k in range(4)]
    lanes = lax.iota(jnp.int32, 16)

    def fire_gather(c, rows, sem):
        for j in range(IDX_ROWS):
            pltpu.async_copy(
                table_hbm.at[idx_all.at[c * IDX_ROWS + j]],
                rows.at[pl.ds(j * 128, 128)],
                sem,
            )

    def drain_gather(rows, sem):
        pltpu.make_async_copy(table_hbm.at[pl.ds(0, CHUNK)], rows, sem).wait()

    def fire_out(outv, c, sem):
        pltpu.async_copy(outv, out_hbm.at[pl.ds(base + c * CHUNK, CHUNK)], sem)

    def drain_out(outv, sem):
        pltpu.make_async_copy(outv, out_hbm.at[pl.ds(base, CHUNK)], sem).wait()

    def compute(rows_v, outv):
        @functools.partial(plsc.parallel_loop, 0, GROUPS)
        def _(grp):
            row_ids = grp * 16 + lanes
            s = [jnp.zeros((16,), jnp.float32) for _ in range(4)]
            q = [jnp.zeros((16,), jnp.float32) for _ in range(4)]
            for j in range(D):
                col = jnp.full((16,), j, jnp.int32)
                g = plsc.load_gather(rows_v, [row_ids, col])
                s[j % 4] = s[j % 4] + g
                q[j % 4] = q[j % 4] + g * g
            stot = (s[0] + s[1]) + (s[2] + s[3])
            qtot = (q[0] + q[1]) + (q[2] + q[3])
            mean = stot * (1.0 / D)
            var = qtot * (1.0 / D) - mean * mean
            inv = _rsqrt(var + EPS)
            for j in range(D):
                col = jnp.full((16,), j, jnp.int32)
                idxj = jnp.full((16,), j % 16, jnp.int32)
                g = plsc.load_gather(rows_v, [row_ids, col])
                gj = jnp.take_along_axis(gam[j // 16], idxj, axis=0)
                bj = jnp.take_along_axis(bet[j // 16], idxj, axis=0)
                o = (g - mean) * inv * gj + bj
                plsc.store_scatter(outv, [row_ids, col], o)

    fire_gather(0, rows0, gsem0)

    def body2(c2, _):
        c0 = 2 * c2
        c1 = c0 + 1
        drain_gather(rows0, gsem0)
        fire_gather(c1, rows1, gsem1)

        @pl.when(c2 > 0)
        def _():
            drain_out(outb0, osem0)

        compute(rows0, outb0)
        fire_out(outb0, c0, osem0)

        drain_gather(rows1, gsem1)

        @pl.when(c2 < NH - 1)
        def _():
            fire_gather(c0 + 2, rows0, gsem0)

        @pl.when(c2 > 0)
        def _():
            drain_out(outb1, osem1)

        compute(rows1, outb1)
        fire_out(outb1, c1, osem1)
        return None

    lax.fori_loop(0, NH, body2, None)
    drain_out(outb0, osem0)
    drain_out(outb1, osem1)


@jax.jit
def _run(x2d, table, gamma, beta):
    mesh = plsc.VectorSubcoreMesh(core_axis_name="c", subcore_axis_name="s")
    kern = pl.kernel(
        _body,
        out_type=jax.ShapeDtypeStruct((N, D), jnp.float32),
        mesh=mesh,
        scratch_types=[
            pltpu.VMEM((IDXR_W, 128), jnp.int32),
            pltpu.VMEM((CHUNK, D), jnp.float32),
            pltpu.VMEM((CHUNK, D), jnp.float32),
            pltpu.VMEM((CHUNK, D), jnp.float32),
            pltpu.VMEM((CHUNK, D), jnp.float32),
            pltpu.VMEM((2, D), jnp.float32),
            pltpu.SemaphoreType.DMA,
            pltpu.SemaphoreType.DMA,
            pltpu.SemaphoreType.DMA,
            pltpu.SemaphoreType.DMA,
        ],
        compiler_params=pltpu.CompilerParams(
            use_tc_tiling_on_sc=False, needs_layout_passes=False
        ),
    )
    return kern(x2d, table, gamma, beta)


def kernel(x, table, gamma, beta):
    x2d = x.reshape(N // 128, 128).astype(jnp.int32)
    out = _run(x2d, table, gamma, beta)
    return out.reshape(B, H, D)


# speedup vs baseline: 3.4635x; 1.0182x over previous
import functools

import jax
import jax.numpy as jnp
from jax import lax
from jax.experimental import pallas as pl
from jax.experimental.pallas import tpu as pltpu
from jax.experimental.pallas import tpu_sc as plsc

NUM_EMBEDDINGS = 1000000
D = 64
B = 4096
H = 200
EPS = 1e-5

NC, NS = 2, 16
NW = NC * NS
B_W = B // NW
NB = 2
CHUNK = NB * H
IW = 80
IR = CHUNK // IW
XR_W = B_W * H // IW
N_CHUNKS = B_W // NB
NH = N_CHUNKS // 2
GROUPS = CHUNK // 16


def _rsqrt(x):
    i = lax.bitcast_convert_type(x, jnp.int32)
    i = jnp.int32(0x5F3759DF) - lax.shift_right_logical(i, 1)
    y = lax.bitcast_convert_type(i, jnp.float32)
    for _ in range(3):
        y = y * (1.5 - 0.5 * x * y * y)
    return y


def _body(x_hbm, table_hbm, gamma_hbm, beta_hbm, out_hbm,
          idx0, idx1, rows0, rows1, outb0, outb1, gb_v,
          isem0, isem1, gsem0, gsem1, osem0, osem1):
    wid = lax.axis_index("s") * NC + lax.axis_index("c")
    b_base = wid * B_W
    xr_base = wid * XR_W

    pltpu.sync_copy(gamma_hbm, gb_v.at[0])
    pltpu.sync_copy(beta_hbm, gb_v.at[1])
    gam = [gb_v[0, pl.ds(16 * k, 16)] for k in range(4)]
    bet = [gb_v[1, pl.ds(16 * k, 16)] for k in range(4)]
    lanes = lax.iota(jnp.int32, 16)

    def fire_idx(c, idx_c, sem):
        pltpu.async_copy(x_hbm.at[pl.ds(xr_base + c * IR, IR)], idx_c, sem)

    def drain_idx(idx_c, sem):
        pltpu.make_async_copy(x_hbm.at[pl.ds(xr_base, IR)], idx_c, sem).wait()

    def fire_gather(idx_c, rows, sem):
        for r in range(IR):
            pltpu.async_copy(
                table_hbm.at[idx_c.at[r]],
                rows.at[pl.ds(r * IW, IW)],
                sem,
            )

    def drain_gather(rows, sem):
        pltpu.make_async_copy(table_hbm.at[pl.ds(0, CHUNK)], rows, sem).wait()

    def fire_out(outv, c, sem):
        pltpu.async_copy(outv, out_hbm.at[pl.ds(b_base + c * NB, NB)], sem)

    def drain_out(outv, sem):
        pltpu.make_async_copy(outv, out_hbm.at[pl.ds(b_base, NB)], sem).wait()

    def compute(rows_v, outv):
        @functools.partial(plsc.parallel_loop, 0, GROUPS)
        def _(grp):
            row_ids = grp * 16 + lanes
            d0 = jnp.where(row_ids >= H, 1, 0).astype(jnp.int32)
            d1 = row_ids - d0 * H
            s = [jnp.zeros((16,), jnp.float32) for _ in range(4)]
            q = [jnp.zeros((16,), jnp.float32) for _ in range(4)]
            for j in range(D):
                col = jnp.full((16,), j, jnp.int32)
                g = plsc.load_gather(rows_v, [row_ids, col])
                s[j % 4] = s[j % 4] + g
                q[j % 4] = q[j % 4] + g * g
            stot = (s[0] + s[1]) + (s[2] + s[3])
            qtot = (q[0] + q[1]) + (q[2] + q[3])
            mean = stot * (1.0 / D)
            var = qtot * (1.0 / D) - mean * mean
            inv = _rsqrt(var + EPS)
            for j in range(D):
                col = jnp.full((16,), j, jnp.int32)
                idxj = jnp.full((16,), j % 16, jnp.int32)
                g = plsc.load_gather(rows_v, [row_ids, col])
                gj = jnp.take_along_axis(gam[j // 16], idxj, axis=0)
                bj = jnp.take_along_axis(bet[j // 16], idxj, axis=0)
                o = (g - mean) * inv * gj + bj
                plsc.store_scatter(outv, [d0, d1, col], o)

    fire_idx(0, idx0, isem0)
    fire_idx(1, idx1, isem1)
    drain_idx(idx0, isem0)
    fire_gather(idx0, rows0, gsem0)

    def body2(c2, _):
        c0 = 2 * c2
        c1 = c0 + 1
        drain_gather(rows0, gsem0)
        drain_idx(idx1, isem1)
        fire_gather(idx1, rows1, gsem1)

        @pl.when(c2 < NH - 1)
        def _():
            fire_idx(c0 + 2, idx0, isem0)

        @pl.when(c2 > 0)
        def _():
            drain_out(outb0, osem0)

        compute(rows0, outb0)
        fire_out(outb0, c0, osem0)

        drain_gather(rows1, gsem1)

        @pl.when(c2 < NH - 1)
        def _():
            drain_idx(idx0, isem0)
            fire_gather(idx0, rows0, gsem0)
            fire_idx(c1 + 2, idx1, isem1)

        @pl.when(c2 > 0)
        def _():
            drain_out(outb1, osem1)

        compute(rows1, outb1)
        fire_out(outb1, c1, osem1)
        return None

    lax.fori_loop(0, NH, body2, None)
    drain_out(outb0, osem0)
    drain_out(outb1, osem1)


@jax.jit
def _run(x2d, table, gamma, beta):
    mesh = plsc.VectorSubcoreMesh(core_axis_name="c", subcore_axis_name="s")
    kern = pl.kernel(
        _body,
        out_type=jax.ShapeDtypeStruct((B, H, D), jnp.float32),
        mesh=mesh,
        scratch_types=[
            pltpu.VMEM((IR, IW), jnp.int32),
            pltpu.VMEM((IR, IW), jnp.int32),
            pltpu.VMEM((CHUNK, D), jnp.float32),
            pltpu.VMEM((CHUNK, D), jnp.float32),
            pltpu.VMEM((NB, H, D), jnp.float32),
            pltpu.VMEM((NB, H, D), jnp.float32),
            pltpu.VMEM((2, D), jnp.float32),
            pltpu.SemaphoreType.DMA,
            pltpu.SemaphoreType.DMA,
            pltpu.SemaphoreType.DMA,
            pltpu.SemaphoreType.DMA,
            pltpu.SemaphoreType.DMA,
            pltpu.SemaphoreType.DMA,
        ],
        compiler_params=pltpu.CompilerParams(
            use_tc_tiling_on_sc=False, needs_layout_passes=False
        ),
    )
    return kern(x2d, table, gamma, beta)


def kernel(x, table, gamma, beta):
    x2d = x.reshape(B * H // IW, IW).astype(jnp.int32)
    return _run(x2d, table, gamma, beta)
